# baseline (speedup 1.0000x reference)
import functools

import jax
import jax.numpy as jnp
from jax import lax
from jax.experimental import pallas as pl
from jax.experimental.pallas import tpu as pltpu
from jax.experimental.pallas import tpu_sc as plsc

_N_NODES = 1000000
_D = 64
_B = 16384

_info = plsc.get_sparse_core_info()
_NC, _NS, _L = _info.num_cores, _info.num_subcores, _info.num_lanes
_NW = _NC * _NS
_BPW = _B // _NW
_CHUNK = 128
_NCHUNK = _BPW // _CHUNK


def _rsqrt_newton(x):
    i = lax.bitcast_convert_type(x, jnp.int32)
    i = jnp.full_like(i, 0x5F3759DF) - lax.shift_right_logical(i, 1)
    y = lax.bitcast_convert_type(i, jnp.float32)
    half_x = x * jnp.float32(0.5)
    for _ in range(3):
        y = y * (jnp.float32(1.5) - half_x * y * y)
    return y


def _hsum(x):
    lanes = lax.iota(jnp.int32, _L)
    for k in (1, 2, 4, 8):
        x = x + x.at[lanes ^ k].get(mode="promise_in_bounds")
    return x


@functools.partial(
    pl.kernel,
    mesh=plsc.VectorSubcoreMesh(core_axis_name="c", subcore_axis_name="s"),
    out_type=jax.ShapeDtypeStruct((_B, _D), jnp.float32),
    scratch_types=[
        pltpu.VMEM((_BPW,), jnp.int32),
        pltpu.VMEM((_BPW, _D), jnp.float32),
        pltpu.SemaphoreType.DMA,
    ],
    compiler_params=pltpu.CompilerParams(use_tc_tiling_on_sc=False),
)
def _emb_norm(nodes_hbm, table_hbm, out_hbm, idx_v, rows_v, sem):
    wid = lax.axis_index("s") * _NC + lax.axis_index("c")
    base = wid * _BPW

    pltpu.sync_copy(nodes_hbm.at[pl.ds(base, _BPW)], idx_v)

    copies = []
    for g in range(_NCHUNK):
        cp = pltpu.make_async_copy(
            table_hbm.at[idx_v.at[pl.ds(g * _CHUNK, _CHUNK)]],
            rows_v.at[pl.ds(g * _CHUNK, _CHUNK)],
            sem,
        )
        cp.start()
        copies.append(cp)
    for cp in copies:
        cp.wait()

    def body(r, carry):
        c0 = rows_v[r, pl.ds(0, _L)]
        c1 = rows_v[r, pl.ds(_L, _L)]
        c2 = rows_v[r, pl.ds(2 * _L, _L)]
        c3 = rows_v[r, pl.ds(3 * _L, _L)]
        ss = c0 * c0 + c1 * c1 + c2 * c2 + c3 * c3
        nrm2 = _hsum(ss)
        inv = _rsqrt_newton(jnp.maximum(nrm2, jnp.float32(1e-24)))
        rows_v[r, pl.ds(0, _L)] = c0 * inv
        rows_v[r, pl.ds(_L, _L)] = c1 * inv
        rows_v[r, pl.ds(2 * _L, _L)] = c2 * inv
        rows_v[r, pl.ds(3 * _L, _L)] = c3 * inv
        return carry

    lax.fori_loop(0, _BPW, body, 0)

    pltpu.sync_copy(rows_v, out_hbm.at[pl.ds(base, _BPW)])


def kernel(nodes, table):
    return _emb_norm(nodes, table)
